# baseline (speedup 1.0000x reference)
import functools

import jax
import jax.numpy as jnp
from jax import lax
from jax.experimental import pallas as pl
from jax.experimental.pallas import tpu as pltpu
from jax.experimental.pallas import tpu_sc as plsc


@functools.lru_cache(maxsize=None)
def _make_sc_gather(vocab, hidden, batch):
    info = plsc.get_sparse_core_info()
    nw = info.num_cores * info.num_subcores
    assert batch % nw == 0 and (batch // nw) % 8 == 0
    b_per_w = batch // nw
    mesh = plsc.VectorSubcoreMesh(core_axis_name="c", subcore_axis_name="s")

    @functools.partial(
        pl.kernel,
        mesh=mesh,
        out_type=jax.ShapeDtypeStruct((batch, hidden), jnp.float32),
        scratch_types=[
            pltpu.VMEM((b_per_w,), jnp.int32),
            pltpu.VMEM((b_per_w, hidden), jnp.float32),
            pltpu.SemaphoreType.DMA,
        ],
        compiler_params=pltpu.CompilerParams(use_tc_tiling_on_sc=False),
    )
    def gather_k(table_hbm, idx_hbm, out_hbm, idx_v, rows_v, sem):
        wid = lax.axis_index("s") * info.num_cores + lax.axis_index("c")
        base = wid * b_per_w
        pltpu.sync_copy(idx_hbm.at[pl.ds(base, b_per_w)], idx_v)
        pltpu.async_copy(table_hbm.at[idx_v], rows_v, sem).wait()
        pltpu.sync_copy(rows_v, out_hbm.at[pl.ds(base, b_per_w)])

    return gather_k


def _proj_body(h_ref, w_ref, b_ref, out_ref):
    acc = lax.dot_general(
        h_ref[...], w_ref[...],
        (((1,), (1,)), ((), ())),
        preferred_element_type=jnp.float32,
    )
    out_ref[...] = acc + b_ref[...]


@functools.lru_cache(maxsize=None)
def _make_tc_proj(vocab, hidden, batch, tile_v):
    grid = (vocab + tile_v - 1) // tile_v
    return pl.pallas_call(
        _proj_body,
        grid=(grid,),
        in_specs=[
            pl.BlockSpec((batch, hidden), lambda i: (0, 0)),
            pl.BlockSpec((tile_v, hidden), lambda i: (i, 0)),
            pl.BlockSpec((1, tile_v), lambda i: (0, i)),
        ],
        out_specs=pl.BlockSpec((batch, tile_v), lambda i: (0, i)),
        out_shape=jax.ShapeDtypeStruct((batch, vocab), jnp.float32),
    )


def kernel(x, emb_table, lin_w, lin_b):
    vocab, hidden = emb_table.shape
    batch = x.shape[0]
    h = _make_sc_gather(vocab, hidden, batch)(emb_table, x.astype(jnp.int32))
    proj = _make_tc_proj(vocab, hidden, batch, 512)
    return proj(h, lin_w, lin_b.reshape(1, vocab))


# speedup vs baseline: 1.1500x; 1.1500x over previous
import functools

import jax
import jax.numpy as jnp
from jax import lax
from jax.experimental import pallas as pl
from jax.experimental.pallas import tpu as pltpu
from jax.experimental.pallas import tpu_sc as plsc


@functools.lru_cache(maxsize=None)
def _make_sc_gather(vocab, hidden, batch):
    info = plsc.get_sparse_core_info()
    nw = info.num_cores * info.num_subcores
    assert batch % nw == 0 and (batch // nw) % 8 == 0
    b_per_w = batch // nw
    mesh = plsc.VectorSubcoreMesh(core_axis_name="c", subcore_axis_name="s")

    @functools.partial(
        pl.kernel,
        mesh=mesh,
        out_type=jax.ShapeDtypeStruct((batch, hidden), jnp.float32),
        scratch_types=[
            pltpu.VMEM((b_per_w,), jnp.int32),
            pltpu.VMEM((b_per_w, hidden), jnp.float32),
            pltpu.SemaphoreType.DMA,
        ],
        compiler_params=pltpu.CompilerParams(use_tc_tiling_on_sc=False),
    )
    def gather_k(table_hbm, idx_hbm, out_hbm, idx_v, rows_v, sem):
        wid = lax.axis_index("s") * info.num_cores + lax.axis_index("c")
        base = wid * b_per_w
        pltpu.sync_copy(idx_hbm.at[pl.ds(base, b_per_w)], idx_v)
        pltpu.async_copy(table_hbm.at[idx_v], rows_v, sem).wait()
        pltpu.sync_copy(rows_v, out_hbm.at[pl.ds(base, b_per_w)])

    return gather_k


def _proj_body(h_ref, w_ref, b_ref, out_ref):
    acc = lax.dot_general(
        h_ref[...], w_ref[...],
        (((1,), (1,)), ((), ())),
        preferred_element_type=jnp.float32,
    )
    out_ref[...] = acc + b_ref[...]


@functools.lru_cache(maxsize=None)
def _make_tc_proj(vocab, hidden, batch, tile_v):
    grid = (vocab + tile_v - 1) // tile_v
    return pl.pallas_call(
        _proj_body,
        grid=(grid,),
        in_specs=[
            pl.BlockSpec((batch, hidden), lambda i: (0, 0)),
            pl.BlockSpec((tile_v, hidden), lambda i: (i, 0)),
            pl.BlockSpec((1, tile_v), lambda i: (0, i)),
        ],
        out_specs=pl.BlockSpec((batch, tile_v), lambda i: (0, i)),
        out_shape=jax.ShapeDtypeStruct((batch, vocab), jnp.float32),
        compiler_params=pltpu.CompilerParams(
            vmem_limit_bytes=100 * 1024 * 1024,
        ),
    )


def kernel(x, emb_table, lin_w, lin_b):
    vocab, hidden = emb_table.shape
    batch = x.shape[0]
    h = _make_sc_gather(vocab, hidden, batch)(emb_table, x.astype(jnp.int32))
    proj = _make_tc_proj(vocab, hidden, batch, 4096)
    return proj(h, lin_w, lin_b.reshape(1, vocab))


# speedup vs baseline: 1.2825x; 1.1153x over previous
import functools

import jax
import jax.numpy as jnp
from jax import lax
from jax.experimental import pallas as pl
from jax.experimental.pallas import tpu as pltpu
from jax.experimental.pallas import tpu_sc as plsc


@functools.lru_cache(maxsize=None)
def _make_sc_gather(vocab, hidden, batch):
    info = plsc.get_sparse_core_info()
    nw = info.num_cores * info.num_subcores
    assert batch % nw == 0 and (batch // nw) % 8 == 0
    b_per_w = batch // nw
    mesh = plsc.VectorSubcoreMesh(core_axis_name="c", subcore_axis_name="s")

    @functools.partial(
        pl.kernel,
        mesh=mesh,
        out_type=jax.ShapeDtypeStruct((batch, hidden), jnp.float32),
        scratch_types=[
            pltpu.VMEM((b_per_w,), jnp.int32),
            pltpu.VMEM((b_per_w, hidden), jnp.float32),
            pltpu.SemaphoreType.DMA,
        ],
        compiler_params=pltpu.CompilerParams(use_tc_tiling_on_sc=False),
    )
    def gather_k(table_hbm, idx_hbm, out_hbm, idx_v, rows_v, sem):
        wid = lax.axis_index("s") * info.num_cores + lax.axis_index("c")
        base = wid * b_per_w
        pltpu.sync_copy(idx_hbm.at[pl.ds(base, b_per_w)], idx_v)
        pltpu.async_copy(table_hbm.at[idx_v], rows_v, sem).wait()
        pltpu.sync_copy(rows_v, out_hbm.at[pl.ds(base, b_per_w)])

    return gather_k


_NBUF = 4


@functools.lru_cache(maxsize=None)
def _make_tc_proj(vocab, hidden, batch, tile_v):
    grid = (vocab + tile_v - 1) // tile_v
    last_cols = vocab - (grid - 1) * tile_v
    tail_aligned = (last_cols // 128) * 128
    rem = last_cols - tail_aligned
    assert grid > _NBUF and tile_v % 128 == 0 and rem > 0

    def body(h_ref, w_ref, b_ref, out_hbm, rem_ref, buf, sems):
        i = pl.program_id(0)
        slot = lax.rem(i, _NBUF)

        def full_copy(step, s):
            return pltpu.make_async_copy(
                buf.at[s],
                out_hbm.at[:, pl.ds(step * tile_v, tile_v)],
                sems.at[s],
            )

        def tail_copy(s):
            return pltpu.make_async_copy(
                buf.at[s, :, pl.ds(0, tail_aligned)],
                out_hbm.at[:, pl.ds((grid - 1) * tile_v, tail_aligned)],
                sems.at[s],
            )

        @pl.when(i >= _NBUF)
        def _():
            full_copy(i - _NBUF, slot).wait()

        acc = lax.dot_general(
            h_ref[...], w_ref[...],
            (((1,), (1,)), ((), ())),
            preferred_element_type=jnp.float32,
        )
        vals = acc + b_ref[0]
        buf[slot] = vals

        @pl.when(i < grid - 1)
        def _():
            full_copy(i, slot).start()

        @pl.when(i == grid - 1)
        def _():
            rem_ref[...] = vals[:, tail_aligned:tail_aligned + rem]
            tail_copy((grid - 1) % _NBUF).start()
            for step in range(grid - _NBUF, grid - 1):
                full_copy(step, step % _NBUF).wait()
            tail_copy((grid - 1) % _NBUF).wait()

    return pl.pallas_call(
        body,
        grid=(grid,),
        in_specs=[
            pl.BlockSpec((batch, hidden), lambda i: (0, 0)),
            pl.BlockSpec((tile_v, hidden), lambda i: (i, 0)),
            pl.BlockSpec((1, 1, tile_v), lambda i: (i, 0, 0)),
        ],
        out_specs=[
            pl.BlockSpec(memory_space=pl.ANY),
            pl.BlockSpec((batch, rem), lambda i: (0, 0)),
        ],
        out_shape=[
            jax.ShapeDtypeStruct((batch, vocab), jnp.float32),
            jax.ShapeDtypeStruct((batch, rem), jnp.float32),
        ],
        scratch_shapes=[
            pltpu.VMEM((_NBUF, batch, tile_v), jnp.float32),
            pltpu.SemaphoreType.DMA((_NBUF,)),
        ],
        compiler_params=pltpu.CompilerParams(
            vmem_limit_bytes=100 * 1024 * 1024,
        ),
    )


def kernel(x, emb_table, lin_w, lin_b):
    vocab, hidden = emb_table.shape
    batch = x.shape[0]
    tile_v = 2048
    grid = (vocab + tile_v - 1) // tile_v
    h = _make_sc_gather(vocab, hidden, batch)(emb_table, x.astype(jnp.int32))
    proj = _make_tc_proj(vocab, hidden, batch, tile_v)
    b_pad = jnp.pad(lin_b, (0, grid * tile_v - vocab)).reshape(grid, 1, tile_v)
    out, rem_out = proj(h, lin_w, b_pad)
    rem = rem_out.shape[1]
    return lax.dynamic_update_slice(out, rem_out, (0, vocab - rem))


# speedup vs baseline: 2.2395x; 1.7461x over previous
import functools

import jax
import jax.numpy as jnp
from jax import lax
from jax.experimental import pallas as pl
from jax.experimental.pallas import tpu as pltpu
from jax.experimental.pallas import tpu_sc as plsc


@functools.lru_cache(maxsize=None)
def _make_sc_gather(vocab, hidden, batch):
    info = plsc.get_sparse_core_info()
    nw = info.num_cores * info.num_subcores
    assert batch % nw == 0 and (batch // nw) % 8 == 0
    b_per_w = batch // nw
    mesh = plsc.VectorSubcoreMesh(core_axis_name="c", subcore_axis_name="s")

    @functools.partial(
        pl.kernel,
        mesh=mesh,
        out_type=jax.ShapeDtypeStruct((batch, hidden), jnp.float32),
        scratch_types=[
            pltpu.VMEM((b_per_w,), jnp.int32),
            pltpu.VMEM((b_per_w, hidden), jnp.float32),
            pltpu.SemaphoreType.DMA,
        ],
        compiler_params=pltpu.CompilerParams(use_tc_tiling_on_sc=False),
    )
    def gather_k(table_hbm, idx_hbm, out_hbm, idx_v, rows_v, sem):
        wid = lax.axis_index("s") * info.num_cores + lax.axis_index("c")
        base = wid * b_per_w
        pltpu.sync_copy(idx_hbm.at[pl.ds(base, b_per_w)], idx_v)
        pltpu.async_copy(table_hbm.at[idx_v], rows_v, sem).wait()
        pltpu.sync_copy(rows_v, out_hbm.at[pl.ds(base, b_per_w)])

    return gather_k


def _proj_body(h_ref, w_ref, b_ref, out_ref):
    acc = lax.dot_general(
        w_ref[...], h_ref[...],
        (((1,), (1,)), ((), ())),
        preferred_element_type=jnp.float32,
    )
    out_ref[...] = acc + b_ref[...]


@functools.lru_cache(maxsize=None)
def _make_tc_proj(vocab, hidden, batch, tile_v):
    grid = (vocab + tile_v - 1) // tile_v
    return pl.pallas_call(
        _proj_body,
        grid=(grid,),
        in_specs=[
            pl.BlockSpec((batch, hidden), lambda i: (0, 0)),
            pl.BlockSpec((tile_v, hidden), lambda i: (i, 0)),
            pl.BlockSpec((tile_v, 1), lambda i: (i, 0)),
        ],
        out_specs=pl.BlockSpec((tile_v, batch), lambda i: (i, 0)),
        out_shape=jax.ShapeDtypeStruct((vocab, batch), jnp.float32),
        compiler_params=pltpu.CompilerParams(
            vmem_limit_bytes=100 * 1024 * 1024,
        ),
    )


def kernel(x, emb_table, lin_w, lin_b):
    vocab, hidden = emb_table.shape
    batch = x.shape[0]
    h = _make_sc_gather(vocab, hidden, batch)(emb_table, x.astype(jnp.int32))
    proj = _make_tc_proj(vocab, hidden, batch, 2048)
    out_t = proj(h, lin_w, lin_b.reshape(vocab, 1))
    return out_t.T


# speedup vs baseline: 2.6013x; 1.1616x over previous
import functools

import jax
import jax.numpy as jnp
from jax import lax
from jax.experimental import pallas as pl
from jax.experimental.pallas import tpu as pltpu
from jax.experimental.pallas import tpu_sc as plsc


@functools.lru_cache(maxsize=None)
def _make_sc_gather(vocab, hidden, batch):
    info = plsc.get_sparse_core_info()
    nw = info.num_cores * info.num_subcores
    assert batch % nw == 0 and (batch // nw) % 8 == 0
    b_per_w = batch // nw
    mesh = plsc.VectorSubcoreMesh(core_axis_name="c", subcore_axis_name="s")

    @functools.partial(
        pl.kernel,
        mesh=mesh,
        out_type=jax.ShapeDtypeStruct((batch, hidden), jnp.float32),
        scratch_types=[
            pltpu.VMEM((b_per_w,), jnp.int32),
            pltpu.VMEM((b_per_w, hidden), jnp.float32),
            pltpu.SemaphoreType.DMA,
        ],
        compiler_params=pltpu.CompilerParams(use_tc_tiling_on_sc=False),
    )
    def gather_k(table_hbm, idx_hbm, out_hbm, idx_v, rows_v, sem):
        wid = lax.axis_index("s") * info.num_cores + lax.axis_index("c")
        base = wid * b_per_w
        pltpu.sync_copy(idx_hbm.at[pl.ds(base, b_per_w)], idx_v)
        pltpu.async_copy(table_hbm.at[idx_v], rows_v, sem).wait()
        pltpu.sync_copy(rows_v, out_hbm.at[pl.ds(base, b_per_w)])

    return gather_k


def _proj_body(h_ref, wt_ref, b_ref, out_ref):
    acc = lax.dot_general(
        wt_ref[...], h_ref[...],
        (((0,), (1,)), ((), ())),
        preferred_element_type=jnp.float32,
    )
    out_ref[...] = acc + b_ref[...]


@functools.lru_cache(maxsize=None)
def _make_tc_proj(vocab, hidden, batch, tile_v):
    grid = (vocab + tile_v - 1) // tile_v
    return pl.pallas_call(
        _proj_body,
        grid=(grid,),
        in_specs=[
            pl.BlockSpec((batch, hidden), lambda i: (0, 0)),
            pl.BlockSpec((hidden, tile_v), lambda i: (0, i)),
            pl.BlockSpec((tile_v, 1), lambda i: (i, 0)),
        ],
        out_specs=pl.BlockSpec((tile_v, batch), lambda i: (i, 0)),
        out_shape=jax.ShapeDtypeStruct((vocab, batch), jnp.float32),
        compiler_params=pltpu.CompilerParams(
            vmem_limit_bytes=100 * 1024 * 1024,
        ),
    )


def kernel(x, emb_table, lin_w, lin_b):
    vocab, hidden = emb_table.shape
    batch = x.shape[0]
    emb2 = emb_table + jnp.float32(0.0)
    h = _make_sc_gather(vocab, hidden, batch)(emb2, x.astype(jnp.int32))
    proj = _make_tc_proj(vocab, hidden, batch, 2048)
    out_t = proj(h, lin_w.T, lin_b.reshape(vocab, 1))
    return out_t.T


# speedup vs baseline: 3.3684x; 1.2949x over previous
import functools

import jax
import jax.numpy as jnp
from jax import lax
from jax.experimental import pallas as pl
from jax.experimental.pallas import tpu as pltpu
from jax.experimental.pallas import tpu_sc as plsc


@functools.lru_cache(maxsize=None)
def _make_sc_gather(vocab, hidden, batch):
    info = plsc.get_sparse_core_info()
    nw = info.num_cores * info.num_subcores
    assert batch % nw == 0 and (batch // nw) % 8 == 0
    b_per_w = batch // nw
    mesh = plsc.VectorSubcoreMesh(core_axis_name="c", subcore_axis_name="s")

    @functools.partial(
        pl.kernel,
        mesh=mesh,
        out_type=jax.ShapeDtypeStruct((batch, hidden), jnp.float32),
        scratch_types=[
            pltpu.VMEM((b_per_w,), jnp.int32),
            pltpu.VMEM((b_per_w, hidden), jnp.float32),
            pltpu.SemaphoreType.DMA,
        ],
        compiler_params=pltpu.CompilerParams(use_tc_tiling_on_sc=False),
    )
    def gather_k(table_hbm, idx_hbm, out_hbm, idx_v, rows_v, sem):
        wid = lax.axis_index("s") * info.num_cores + lax.axis_index("c")
        base = wid * b_per_w
        pltpu.sync_copy(idx_hbm.at[pl.ds(base, b_per_w)], idx_v)
        pltpu.async_copy(table_hbm.at[idx_v], rows_v, sem).wait()
        pltpu.sync_copy(rows_v, out_hbm.at[pl.ds(base, b_per_w)])

    return gather_k


def _proj_body(h_ref, wt_ref, b_ref, out_ref):
    acc = lax.dot_general(
        wt_ref[...], h_ref[...],
        (((0,), (1,)), ((), ())),
        preferred_element_type=jnp.float32,
    )
    tv, b = acc.shape
    acc3 = acc.reshape(tv // 128, 128, b) + b_ref[...].reshape(tv // 128, 128, 1)
    out_ref[...] = acc3.reshape(tv, b)


@functools.lru_cache(maxsize=None)
def _make_tc_proj(vocab, hidden, batch, tile_v):
    grid = (vocab + tile_v - 1) // tile_v
    return pl.pallas_call(
        _proj_body,
        grid=(grid,),
        in_specs=[
            pl.BlockSpec((batch, hidden), lambda i: (0, 0)),
            pl.BlockSpec((hidden, tile_v), lambda i: (0, i)),
            pl.BlockSpec((tile_v // 128, 128), lambda i: (i, 0)),
        ],
        out_specs=pl.BlockSpec((tile_v, batch), lambda i: (i, 0)),
        out_shape=jax.ShapeDtypeStruct((vocab, batch), jnp.float32),
        compiler_params=pltpu.CompilerParams(
            vmem_limit_bytes=100 * 1024 * 1024,
        ),
    )


def kernel(x, emb_table, lin_w, lin_b):
    vocab, hidden = emb_table.shape
    batch = x.shape[0]
    tile_v = 2048
    grid = (vocab + tile_v - 1) // tile_v
    emb_lin = lax.optimization_barrier(emb_table.reshape(vocab * hidden))
    emb2 = emb_lin.reshape(vocab, hidden)
    h = _make_sc_gather(vocab, hidden, batch)(emb2, x.astype(jnp.int32))
    proj = _make_tc_proj(vocab, hidden, batch, tile_v)
    b2 = jnp.pad(lin_b, (0, grid * tile_v - vocab)).reshape(grid * tile_v // 128, 128)
    out_t = proj(h, lin_w.T, b2)
    return out_t.T
